# baseline (device time: 53791 ns/iter reference)
import jax
import jax.numpy as jnp
from jax import lax
from jax.experimental import pallas as pl
from jax.experimental.pallas import tpu as pltpu

N_Z = 4
N_RING = 8


def kernel(x):
    m_per, n = x.shape
    s_rows = m_per // N_RING

    def body(x_ref, out_ref, stage_ref,
             z_send, z_recv, cw_send, cw_recv, ccw_send, ccw_recv, exit_sem):
        my_x = lax.axis_index("x")
        my_y = lax.axis_index("y")
        my_z = lax.axis_index("z")

        my_r = jnp.where(my_x == 0, my_y, 7 - my_y)

        def ring_coords(p):
            return jnp.where(p < 4, 0, 1), jnp.where(p < 4, p, 7 - p)

        rx, ry = ring_coords((my_r + 1) % N_RING)
        lx, ly = ring_coords((my_r - 1) % N_RING)
        RIGHT = (rx, ry, my_z)
        LEFT = (lx, ly, my_z)
        z_up = jnp.minimum(my_z + 1, N_Z - 1)
        z_dn = jnp.maximum(my_z - 1, 0)
        UP = (my_x, my_y, z_up)
        DN = (my_x, my_y, z_dn)

        def rcopy(src, dst, ssem, rsem, dev):
            return pltpu.make_async_remote_copy(
                src_ref=src, dst_ref=dst, send_sem=ssem, recv_sem=rsem,
                device_id=dev, device_id_type=pl.DeviceIdType.MESH,
            )

        def rowref(ch, origin):
            return out_ref.at[pl.ds(ch * m_per + origin * s_rows, s_rows), :]

        def ordz(ch, zz):
            m = jnp.abs(ch - zz)
            o = (jnp.minimum(m - 1, zz) + jnp.minimum(m - 1, N_Z - 1 - zz)
                 + jnp.where((ch > zz) & (m <= zz), 1, 0))
            return jnp.clip(o, 0, 2)

        chunk_of = [
            jnp.where(my_z > 0, my_z - 1, 1),
            jnp.where(my_z <= 1, 2, jnp.where(my_z == 2, 3, 1)),
            jnp.where(my_z <= 1, 3, 0),
        ]

        cw_hops = [4, 3, 4]
        ccw_hops = [3, 4, 3]

        partners = [
            RIGHT, LEFT,
            (my_x, my_y, (my_z + 1) % N_Z),
            (my_x, my_y, (my_z - 1) % N_Z),
        ]
        barrier = pltpu.get_barrier_semaphore()
        for pid in partners:
            pl.semaphore_signal(
                barrier, inc=1, device_id=pid,
                device_id_type=pl.DeviceIdType.MESH,
            )
        pl.semaphore_wait(barrier, 4)

        stage_ref[:, :] = x_ref[pl.ds(my_r * s_rows, s_rows), :]

        for t in range(1, N_Z):
            @pl.when(my_z + t <= N_Z - 1)
            def _(t=t):
                tz = jnp.minimum(my_z + t, N_Z - 1)
                rcopy(stage_ref, rowref(my_z, my_r),
                      z_send.at[t - 1], z_recv.at[ordz(my_z, tz)],
                      (my_x, my_y, tz)).start()

            @pl.when(my_z - t >= 0)
            def _(t=t):
                tz = jnp.maximum(my_z - t, 0)
                rcopy(stage_ref, rowref(my_z, my_r),
                      z_send.at[2 + t], z_recv.at[ordz(my_z, tz)],
                      (my_x, my_y, tz)).start()

        out_ref[pl.ds(my_z * m_per, m_per), :] = x_ref[:, :]

        for d in range(3):
            ch = chunk_of[d]
            rcopy(stage_ref, rowref(ch, my_r),
                  z_send.at[0], z_recv.at[d], UP).wait_recv()

            rcopy(rowref(ch, my_r), rowref(ch, my_r),
                  cw_send.at[d], cw_recv.at[d * 4], RIGHT).start()
            rcopy(rowref(ch, my_r), rowref(ch, my_r),
                  ccw_send.at[d], ccw_recv.at[d * 4], LEFT).start()

        for j in range(4):
            for d in range(3):
                ch = chunk_of[d]
                if j < cw_hops[d]:
                    o_in = (my_r - j - 1) % N_RING
                    rcopy(rowref(ch, o_in), rowref(ch, o_in),
                          cw_send.at[d], cw_recv.at[d * 4 + j], RIGHT).wait_recv()
                    if j + 1 < cw_hops[d]:
                        rcopy(rowref(ch, o_in), rowref(ch, o_in),
                              cw_send.at[d], cw_recv.at[d * 4 + j], RIGHT).wait_send()
                        rcopy(rowref(ch, o_in), rowref(ch, o_in),
                              cw_send.at[d], cw_recv.at[d * 4 + j + 1], RIGHT).start()
                if j < ccw_hops[d]:
                    o_in = (my_r + j + 1) % N_RING
                    rcopy(rowref(ch, o_in), rowref(ch, o_in),
                          ccw_send.at[d], ccw_recv.at[d * 4 + j], LEFT).wait_recv()
                    if j + 1 < ccw_hops[d]:
                        rcopy(rowref(ch, o_in), rowref(ch, o_in),
                              ccw_send.at[d], ccw_recv.at[d * 4 + j], LEFT).wait_send()
                        rcopy(rowref(ch, o_in), rowref(ch, o_in),
                              ccw_send.at[d], ccw_recv.at[d * 4 + j + 1], LEFT).start()

        for t in range(1, N_Z):
            @pl.when(my_z + t <= N_Z - 1)
            def _(t=t):
                rcopy(stage_ref, rowref(my_z, my_r), z_send.at[t - 1],
                      z_recv.at[0], UP).wait_send()

            @pl.when(my_z - t >= 0)
            def _(t=t):
                rcopy(stage_ref, rowref(my_z, my_r), z_send.at[2 + t],
                      z_recv.at[0], DN).wait_send()

        for d in range(3):
            ch = chunk_of[d]
            jl = cw_hops[d] - 1
            o_in = (my_r - jl - 1) % N_RING
            rcopy(rowref(ch, o_in), rowref(ch, o_in),
                  cw_send.at[d], cw_recv.at[d * 4 + jl], RIGHT).wait_send()
            jl = ccw_hops[d] - 1
            o_in = (my_r + jl + 1) % N_RING
            rcopy(rowref(ch, o_in), rowref(ch, o_in),
                  ccw_send.at[d], ccw_recv.at[d * 4 + jl], LEFT).wait_send()

        for pid in partners:
            pl.semaphore_signal(
                exit_sem, inc=1, device_id=pid,
                device_id_type=pl.DeviceIdType.MESH,
            )
        pl.semaphore_wait(exit_sem, 4)

    return pl.pallas_call(
        body,
        out_shape=jax.ShapeDtypeStruct((N_Z * m_per, n), x.dtype),
        in_specs=[pl.BlockSpec(memory_space=pltpu.VMEM)],
        out_specs=pl.BlockSpec(memory_space=pltpu.VMEM),
        scratch_shapes=[
            pltpu.VMEM((s_rows, n), x.dtype),
            pltpu.SemaphoreType.DMA((6,)),
            pltpu.SemaphoreType.DMA((3,)),
            pltpu.SemaphoreType.DMA((3,)),
            pltpu.SemaphoreType.DMA((12,)),
            pltpu.SemaphoreType.DMA((3,)),
            pltpu.SemaphoreType.DMA((12,)),
            pltpu.SemaphoreType.REGULAR,
        ],
        compiler_params=pltpu.CompilerParams(collective_id=0),
    )(x)
